# Initial kernel scaffold; baseline (speedup 1.0000x reference)
import jax
import jax.numpy as jnp
from jax.experimental import pallas as pl


def kernel(x, embed_mu, embed_logvar):
    raise NotImplementedError("write your pallas kernel here")


# speedup vs baseline: 2.6461x; 2.6461x over previous
import functools

import jax
import jax.numpy as jnp
from jax import lax
from jax.experimental import pallas as pl
from jax.experimental.pallas import tpu as pltpu
from jax.experimental.pallas import tpu_sc as plsc

NUM_CLUSTERS = 1000000
Z_N = 16
BATCH = 16384
HIST = 200

_INFO = plsc.get_sparse_core_info()
_NC = _INFO.num_cores
_NS = _INFO.num_subcores
_NW = _NC * _NS

_N = BATCH * HIST
_PER_W = _N // _NW
_CHUNK = 2048
_STEPS = _PER_W // _CHUNK


def _gather_body(x_hbm, mu_hbm, lv_hbm, out_mu, out_lv,
                 idx_v, mu_v, lv_v, sem_mu, sem_lv):
    wid = lax.axis_index("s") * _NC + lax.axis_index("c")
    base = wid * _PER_W

    def step(i, _):
        off = base + i * _CHUNK
        pltpu.sync_copy(x_hbm.at[pl.ds(off, _CHUNK)], idx_v)
        cp_mu = pltpu.async_copy(mu_hbm.at[idx_v], mu_v, sem_mu)
        cp_lv = pltpu.async_copy(lv_hbm.at[idx_v], lv_v, sem_lv)
        cp_mu.wait()
        cp_lv.wait()
        pltpu.sync_copy(mu_v, out_mu.at[pl.ds(off, _CHUNK)])
        pltpu.sync_copy(lv_v, out_lv.at[pl.ds(off, _CHUNK)])
        return 0

    lax.fori_loop(0, _STEPS, step, 0)


@functools.partial(jax.jit, donate_argnums=())
def _run(x_flat, embed_mu, embed_logvar):
    mesh = plsc.VectorSubcoreMesh(core_axis_name="c", subcore_axis_name="s")
    out_mu, out_lv = pl.kernel(
        _gather_body,
        out_type=(
            jax.ShapeDtypeStruct((_N, Z_N), jnp.float32),
            jax.ShapeDtypeStruct((_N, Z_N), jnp.float32),
        ),
        mesh=mesh,
        scratch_types=[
            pltpu.VMEM((_CHUNK,), jnp.int32),
            pltpu.VMEM((_CHUNK, Z_N), jnp.float32),
            pltpu.VMEM((_CHUNK, Z_N), jnp.float32),
            pltpu.SemaphoreType.DMA,
            pltpu.SemaphoreType.DMA,
        ],
        compiler_params=pltpu.CompilerParams(use_tc_tiling_on_sc=False),
    )(x_flat, embed_mu, embed_logvar)
    return out_mu, out_lv


def kernel(x, embed_mu, embed_logvar):
    x_flat = x.reshape(-1).astype(jnp.int32)
    out_mu, out_lv = _run(x_flat, embed_mu, embed_logvar)
    mu = out_mu.reshape(BATCH, HIST, Z_N)
    logvar = out_lv.reshape(BATCH, HIST, Z_N)
    return (mu, mu, logvar)


# speedup vs baseline: 2.6779x; 1.0120x over previous
import functools

import jax
import jax.numpy as jnp
from jax import lax
from jax.experimental import pallas as pl
from jax.experimental.pallas import tpu as pltpu
from jax.experimental.pallas import tpu_sc as plsc

NUM_CLUSTERS = 1000000
Z_N = 16
BATCH = 16384
HIST = 200

_INFO = plsc.get_sparse_core_info()
_NC = _INFO.num_cores
_NS = _INFO.num_subcores
_NW = _NC * _NS

_N = BATCH * HIST
_PER_W = _N // _NW
_CHUNK = 1600
_STEPS = _PER_W // _CHUNK
_PAIRS = _STEPS // 2


def _gather_body(x_hbm, mu_hbm, lv_hbm, out_mu, out_lv,
                 idx0, idx1, mu0, mu1, lv0, lv1,
                 sg_mu0, sg_mu1, sg_lv0, sg_lv1,
                 sw_mu0, sw_mu1, sw_lv0, sw_lv1):
    idx = (idx0, idx1)
    mu_v = (mu0, mu1)
    lv_v = (lv0, lv1)
    sg_mu = (sg_mu0, sg_mu1)
    sg_lv = (sg_lv0, sg_lv1)
    sw_mu = (sw_mu0, sw_mu1)
    sw_lv = (sw_lv0, sw_lv1)

    wid = lax.axis_index("s") * _NC + lax.axis_index("c")
    base = wid * _PER_W

    def start_gathers(b, c):
        off = base + c * _CHUNK
        pltpu.sync_copy(x_hbm.at[pl.ds(off, _CHUNK)], idx[b])
        pltpu.async_copy(mu_hbm.at[idx[b]], mu_v[b], sg_mu[b])
        pltpu.async_copy(lv_hbm.at[idx[b]], lv_v[b], sg_lv[b])

    def wait_gathers(b):
        pltpu.make_async_copy(mu_hbm.at[idx[b]], mu_v[b], sg_mu[b]).wait()
        pltpu.make_async_copy(lv_hbm.at[idx[b]], lv_v[b], sg_lv[b]).wait()

    def start_writes(b, c):
        off = base + c * _CHUNK
        pltpu.async_copy(mu_v[b], out_mu.at[pl.ds(off, _CHUNK)], sw_mu[b])
        pltpu.async_copy(lv_v[b], out_lv.at[pl.ds(off, _CHUNK)], sw_lv[b])

    def wait_writes(b):
        pltpu.make_async_copy(mu_v[b], out_mu.at[pl.ds(0, _CHUNK)], sw_mu[b]).wait()
        pltpu.make_async_copy(lv_v[b], out_lv.at[pl.ds(0, _CHUNK)], sw_lv[b]).wait()

    start_gathers(0, 0)

    def pair(j, _):
        c0 = 2 * j
        wait_gathers(0)
        start_writes(0, c0)

        @pl.when(j > 0)
        def _():
            wait_writes(1)

        start_gathers(1, c0 + 1)

        wait_gathers(1)
        start_writes(1, c0 + 1)
        wait_writes(0)

        @pl.when(j < _PAIRS - 1)
        def _():
            start_gathers(0, c0 + 2)

        return 0

    lax.fori_loop(0, _PAIRS, pair, 0)
    wait_writes(1)


@functools.partial(jax.jit, donate_argnums=())
def _run(x_flat, embed_mu, embed_logvar):
    mesh = plsc.VectorSubcoreMesh(core_axis_name="c", subcore_axis_name="s")
    out_mu, out_lv = pl.kernel(
        _gather_body,
        out_type=(
            jax.ShapeDtypeStruct((_N, Z_N), jnp.float32),
            jax.ShapeDtypeStruct((_N, Z_N), jnp.float32),
        ),
        mesh=mesh,
        scratch_types=[
            pltpu.VMEM((_CHUNK,), jnp.int32),
            pltpu.VMEM((_CHUNK,), jnp.int32),
            pltpu.VMEM((_CHUNK, Z_N), jnp.float32),
            pltpu.VMEM((_CHUNK, Z_N), jnp.float32),
            pltpu.VMEM((_CHUNK, Z_N), jnp.float32),
            pltpu.VMEM((_CHUNK, Z_N), jnp.float32),
            pltpu.SemaphoreType.DMA,
            pltpu.SemaphoreType.DMA,
            pltpu.SemaphoreType.DMA,
            pltpu.SemaphoreType.DMA,
            pltpu.SemaphoreType.DMA,
            pltpu.SemaphoreType.DMA,
            pltpu.SemaphoreType.DMA,
            pltpu.SemaphoreType.DMA,
        ],
        compiler_params=pltpu.CompilerParams(use_tc_tiling_on_sc=False),
    )(x_flat, embed_mu, embed_logvar)
    return out_mu, out_lv


def kernel(x, embed_mu, embed_logvar):
    x_flat = x.reshape(-1).astype(jnp.int32)
    out_mu, out_lv = _run(x_flat, embed_mu, embed_logvar)
    mu = out_mu.reshape(BATCH, HIST, Z_N)
    logvar = out_lv.reshape(BATCH, HIST, Z_N)
    return (mu, mu, logvar)


# speedup vs baseline: 3.6993x; 1.3814x over previous
import functools

import jax
import jax.numpy as jnp
from jax import lax
from jax.experimental import pallas as pl
from jax.experimental.pallas import tpu as pltpu
from jax.experimental.pallas import tpu_sc as plsc

NUM_CLUSTERS = 1000000
Z_N = 16
BATCH = 16384
HIST = 200

_INFO = plsc.get_sparse_core_info()
_NC = _INFO.num_cores
_NS = _INFO.num_subcores
_NW = _NC * _NS

_BT = BATCH // _NW
_TC = _BT // 128


def _body(x_hbm, mu_hbm, lv_hbm, out_z, out_mu, out_lv,
          idx_v, mu_v, lv_v, tmu_v, tlv_v, sem_mu, sem_lv):
    wid = lax.axis_index("s") * _NC + lax.axis_index("c")
    bt0 = wid * _TC

    iota = lax.iota(jnp.int32, 16)

    def transpose_block(src_v, dst_v, b16):
        d0 = jnp.full((16,), b16 // 8, jnp.int32)
        l0 = (b16 % 8) * 16
        d1 = iota + l0
        for z in range(Z_N):
            d2 = jnp.full((16,), z, jnp.int32)
            v = plsc.load_gather(src_v, [d0, d1, d2])
            dst_v[z // 8, b16 // 8, z % 8, pl.ds(l0, 16)] = v

    def step(h, _):
        ht = h // 8
        hs = h % 8
        pltpu.sync_copy(x_hbm.at[ht, pl.ds(bt0, _TC), hs, :], idx_v)
        for j in range(_TC):
            pltpu.async_copy(mu_hbm.at[idx_v.at[j]], mu_v.at[j], sem_mu)
            pltpu.async_copy(lv_hbm.at[idx_v.at[j]], lv_v.at[j], sem_lv)
        for j in range(_TC):
            pltpu.make_async_copy(mu_hbm.at[idx_v.at[j]], mu_v.at[j], sem_mu).wait()
            pltpu.make_async_copy(lv_hbm.at[idx_v.at[j]], lv_v.at[j], sem_lv).wait()

        def tblock(b16, _):
            transpose_block(mu_v, tmu_v, b16)
            transpose_block(lv_v, tlv_v, b16)
            return 0

        lax.fori_loop(0, _BT // 16, tblock, 0)

        pltpu.sync_copy(tmu_v, out_z.at[h, :, pl.ds(bt0, _TC), :, :])
        pltpu.sync_copy(tmu_v, out_mu.at[h, :, pl.ds(bt0, _TC), :, :])
        pltpu.sync_copy(tlv_v, out_lv.at[h, :, pl.ds(bt0, _TC), :, :])
        return 0

    lax.fori_loop(0, HIST, step, 0)


@jax.jit
def _run(x5, embed_mu, embed_logvar):
    mesh = plsc.VectorSubcoreMesh(core_axis_name="c", subcore_axis_name="s")
    return pl.kernel(
        _body,
        out_type=(
            jax.ShapeDtypeStruct((HIST, 2, 128, 8, 128), jnp.float32),
            jax.ShapeDtypeStruct((HIST, 2, 128, 8, 128), jnp.float32),
            jax.ShapeDtypeStruct((HIST, 2, 128, 8, 128), jnp.float32),
        ),
        mesh=mesh,
        scratch_types=[
            pltpu.VMEM((_TC, 128), jnp.int32),
            pltpu.VMEM((_TC, 128, Z_N), jnp.float32),
            pltpu.VMEM((_TC, 128, Z_N), jnp.float32),
            pltpu.VMEM((2, _TC, 8, 128), jnp.float32),
            pltpu.VMEM((2, _TC, 8, 128), jnp.float32),
            pltpu.SemaphoreType.DMA,
            pltpu.SemaphoreType.DMA,
        ],
        compiler_params=pltpu.CompilerParams(
            use_tc_tiling_on_sc=False, needs_layout_passes=False),
    )(x5, embed_mu, embed_logvar)


def kernel(x, embed_mu, embed_logvar):
    x5 = x.T.reshape(25, 8, 128, 128).transpose(0, 2, 1, 3).astype(jnp.int32)
    out_z, out_mu, out_lv = _run(x5, embed_mu, embed_logvar)
    perm = (2, 4, 0, 1, 3)
    z = out_z.transpose(perm).reshape(BATCH, HIST, Z_N)
    mu = out_mu.transpose(perm).reshape(BATCH, HIST, Z_N)
    logvar = out_lv.transpose(perm).reshape(BATCH, HIST, Z_N)
    return (z, mu, logvar)


# speedup vs baseline: 8.3188x; 2.2487x over previous
import functools

import jax
import jax.numpy as jnp
from jax import lax
from jax.experimental import pallas as pl
from jax.experimental.pallas import tpu as pltpu
from jax.experimental.pallas import tpu_sc as plsc

NUM_CLUSTERS = 1000000
Z_N = 16
BATCH = 16384
HIST = 200

_INFO = plsc.get_sparse_core_info()
_NC = _INFO.num_cores
_NS = _INFO.num_subcores
_NW = _NC * _NS

_BT = BATCH // _NW
_TC = _BT // 128
_HPAIRS = HIST // 2


def _body(x_hbm, mu_hbm, lv_hbm, out_z, out_mu, out_lv,
          idx0, idx1, mu0, mu1, lv0, lv1, tmu0, tmu1, tlv0, tlv1,
          sgm0, sgm1, sgl0, sgl1, swz0, swz1, swm0, swm1, swl0, swl1):
    idx = (idx0, idx1)
    mu_v = (mu0, mu1)
    lv_v = (lv0, lv1)
    tmu = (tmu0, tmu1)
    tlv = (tlv0, tlv1)
    sgm = (sgm0, sgm1)
    sgl = (sgl0, sgl1)
    swz = (swz0, swz1)
    swm = (swm0, swm1)
    swl = (swl0, swl1)

    wid = lax.axis_index("s") * _NC + lax.axis_index("c")
    bt0 = wid * _TC
    iota = lax.iota(jnp.int32, 16)

    def start_gathers(b, h):
        ht = h // 8
        hs = h % 8
        pltpu.sync_copy(x_hbm.at[ht, pl.ds(bt0, _TC), hs, :], idx[b])
        for j in range(_TC):
            pltpu.async_copy(mu_hbm.at[idx[b].at[j]], mu_v[b].at[j], sgm[b])
            pltpu.async_copy(lv_hbm.at[idx[b].at[j]], lv_v[b].at[j], sgl[b])

    def wait_gathers(b):
        for j in range(_TC):
            pltpu.make_async_copy(mu_hbm.at[idx[b].at[j]], mu_v[b].at[j], sgm[b]).wait()
            pltpu.make_async_copy(lv_hbm.at[idx[b].at[j]], lv_v[b].at[j], sgl[b]).wait()

    def transpose(b):
        @functools.partial(plsc.parallel_loop, 0, _BT // 16, unroll=2)
        def _(b16):
            d0 = jnp.full((16,), b16 // 8, jnp.int32)
            l0 = (b16 % 8) * 16
            d1 = iota + l0
            for z in range(Z_N):
                d2 = jnp.full((16,), z, jnp.int32)
                tmu[b][z // 8, b16 // 8, z % 8, pl.ds(l0, 16)] = (
                    plsc.load_gather(mu_v[b], [d0, d1, d2]))
            for z in range(Z_N):
                d2 = jnp.full((16,), z, jnp.int32)
                tlv[b][z // 8, b16 // 8, z % 8, pl.ds(l0, 16)] = (
                    plsc.load_gather(lv_v[b], [d0, d1, d2]))

    def start_writes(b, h):
        dst = (pl.ds(bt0, _TC),)
        pltpu.async_copy(tmu[b], out_z.at[h, :, pl.ds(bt0, _TC), :, :], swz[b])
        pltpu.async_copy(tmu[b], out_mu.at[h, :, pl.ds(bt0, _TC), :, :], swm[b])
        pltpu.async_copy(tlv[b], out_lv.at[h, :, pl.ds(bt0, _TC), :, :], swl[b])

    def wait_writes(b):
        pltpu.make_async_copy(tmu[b], out_z.at[0, :, pl.ds(bt0, _TC), :, :], swz[b]).wait()
        pltpu.make_async_copy(tmu[b], out_mu.at[0, :, pl.ds(bt0, _TC), :, :], swm[b]).wait()
        pltpu.make_async_copy(tlv[b], out_lv.at[0, :, pl.ds(bt0, _TC), :, :], swl[b]).wait()

    start_gathers(0, 0)

    def pair(j, _):
        h0 = 2 * j
        wait_gathers(0)
        start_gathers(1, h0 + 1)

        @pl.when(j > 0)
        def _():
            wait_writes(0)

        transpose(0)
        start_writes(0, h0)

        wait_gathers(1)

        @pl.when(j < _HPAIRS - 1)
        def _():
            start_gathers(0, h0 + 2)

        @pl.when(j > 0)
        def _():
            wait_writes(1)

        transpose(1)
        start_writes(1, h0 + 1)
        return 0

    lax.fori_loop(0, _HPAIRS, pair, 0)
    wait_writes(0)
    wait_writes(1)


@jax.jit
def _run(x5, embed_mu, embed_logvar):
    mesh = plsc.VectorSubcoreMesh(core_axis_name="c", subcore_axis_name="s")
    dma = pltpu.SemaphoreType.DMA
    return pl.kernel(
        _body,
        out_type=(
            jax.ShapeDtypeStruct((HIST, 2, 128, 8, 128), jnp.float32),
            jax.ShapeDtypeStruct((HIST, 2, 128, 8, 128), jnp.float32),
            jax.ShapeDtypeStruct((HIST, 2, 128, 8, 128), jnp.float32),
        ),
        mesh=mesh,
        scratch_types=(
            [pltpu.VMEM((_TC, 128), jnp.int32)] * 2
            + [pltpu.VMEM((_TC, 128, Z_N), jnp.float32)] * 4
            + [pltpu.VMEM((2, _TC, 8, 128), jnp.float32)] * 4
            + [dma] * 10
        ),
        compiler_params=pltpu.CompilerParams(
            use_tc_tiling_on_sc=False, needs_layout_passes=False),
    )(x5, embed_mu, embed_logvar)


def kernel(x, embed_mu, embed_logvar):
    x5 = x.T.reshape(25, 8, 128, 128).transpose(0, 2, 1, 3).astype(jnp.int32)
    out_z, out_mu, out_lv = _run(x5, embed_mu, embed_logvar)
    perm = (2, 4, 0, 1, 3)
    z = out_z.transpose(perm).reshape(BATCH, HIST, Z_N)
    mu = out_mu.transpose(perm).reshape(BATCH, HIST, Z_N)
    logvar = out_lv.transpose(perm).reshape(BATCH, HIST, Z_N)
    return (z, mu, logvar)
